# baseline (speedup 1.0000x reference)
import jax
import jax.numpy as jnp
from jax.experimental import pallas as pl
from jax.experimental.pallas import tpu as pltpu
from jax.experimental.pallas import tpu_sc as plsc

EPS = 1e-3

_GATHER_WINDOW = 128


def _sc_gather(tok_emb, idx2d, n, d):
    mesh = plsc.VectorSubcoreMesh(core_axis_name="core",
                                  subcore_axis_name="subcore")

    @pl.kernel(out_type=jax.ShapeDtypeStruct((n, d), tok_emb.dtype),
               mesh=mesh)
    def gather_kernel(x_hbm, i_hbm, o_hbm):
        def body(i_vmem, o_vmem):
            pltpu.sync_copy(x_hbm.at[i_vmem.at[0]], o_vmem)

        pltpu.emit_pipeline(
            body,
            grid=(n // _GATHER_WINDOW,),
            in_specs=[pl.BlockSpec((1, _GATHER_WINDOW),
                                   index_map=lambda i: (0, i))],
            out_specs=[pl.BlockSpec((_GATHER_WINDOW, d),
                                    index_map=lambda i: (i, 0))],
            core_axis_name=("core", "subcore"),
            dimension_semantics=(pltpu.PARALLEL,),
        )(i_hbm, o_hbm)

    return gather_kernel(tok_emb, idx2d)


def _dense_body(x_ref, pos_ref, gamma_ref, beta_ref, wt_ref, b_ref, o_ref):
    d = pos_ref.shape[2]
    x = x_ref[:, :d] + pos_ref[0]
    mean = jnp.mean(x, axis=1, keepdims=True)
    xc = x - mean
    var = jnp.mean(xc * xc, axis=1, keepdims=True)
    xn = xc * jax.lax.rsqrt(var + EPS)
    xn = xn * gamma_ref[...] + beta_ref[...]
    logits = jax.lax.dot_general(
        wt_ref[...], xn,
        (((1,), (1,)), ((), ())),
        preferred_element_type=jnp.float32,
    ) + b_ref[...]
    o_ref[...] = logits[None]


def _tc_dense(x_t, pos_emb3, gamma, beta, Wt, b_col, batch, seq, d, v,
              interpret=False):
    dx = x_t.shape[1]
    return pl.pallas_call(
        _dense_body,
        grid=(seq,),
        in_specs=[
            pl.BlockSpec((batch, dx), lambda s: (s, 0)),
            pl.BlockSpec((1, 1, d), lambda s: (s, 0, 0)),
            pl.BlockSpec((1, d), lambda s: (0, 0)),
            pl.BlockSpec((1, d), lambda s: (0, 0)),
            pl.BlockSpec((v, d), lambda s: (0, 0)),
            pl.BlockSpec((v, 1), lambda s: (0, 0)),
        ],
        out_specs=pl.BlockSpec((1, v, batch), lambda s: (s, 0, 0)),
        out_shape=jax.ShapeDtypeStruct((seq, v, batch), jnp.float32),
        compiler_params=pltpu.CompilerParams(
            dimension_semantics=("arbitrary",),
        ),
        interpret=interpret,
    )(x_t, pos_emb3, gamma, beta, Wt, b_col)


def kernel(inputs, tok_emb, pos_emb, gamma, beta, W, b):
    batch, seq = inputs.shape
    vocab, d = tok_emb.shape
    v_out = W.shape[1]
    n = batch * seq

    idx2d = inputs.T.reshape(1, n).astype(jnp.int32)
    d_pad = 128
    tok_pad = jnp.pad(tok_emb, ((0, 0), (0, d_pad - d)))
    x_t = _sc_gather(tok_pad, idx2d, n, d_pad)

    logits_t = _tc_dense(x_t, pos_emb.reshape(seq, 1, d),
                         gamma.reshape(1, d), beta.reshape(1, d), W.T,
                         b.reshape(v_out, 1), batch, seq, d, v_out)
    return jnp.transpose(logits_t, (2, 0, 1))


# speedup vs baseline: 1.1432x; 1.1432x over previous
import jax
import jax.numpy as jnp
from jax.experimental import pallas as pl
from jax.experimental.pallas import tpu as pltpu
from jax.experimental.pallas import tpu_sc as plsc

EPS = 1e-3

_GATHER_WINDOW = 128


def _sc_gather(tok_emb, idx2d, n, d):
    mesh = plsc.VectorSubcoreMesh(core_axis_name="core",
                                  subcore_axis_name="subcore")

    @pl.kernel(out_type=jax.ShapeDtypeStruct((n, d), tok_emb.dtype),
               mesh=mesh,
               scratch_types=[pltpu.VMEM_SHARED(tok_emb.shape,
                                                tok_emb.dtype),
                              pltpu.SemaphoreType.DMA])
    def gather_kernel(x_hbm, i_hbm, o_hbm, tab_ref, sem):
        @pl.when(jax.lax.axis_index("subcore") == 0)
        def _():
            pltpu.async_copy(x_hbm, tab_ref, sem).wait()

        plsc.subcore_barrier()

        def body(i_vmem, o_vmem):
            pltpu.sync_copy(tab_ref.at[i_vmem.at[0]], o_vmem)

        pltpu.emit_pipeline(
            body,
            grid=(n // _GATHER_WINDOW,),
            in_specs=[pl.BlockSpec((1, _GATHER_WINDOW),
                                   index_map=lambda i: (0, i))],
            out_specs=[pl.BlockSpec((_GATHER_WINDOW, d),
                                    index_map=lambda i: (i, 0))],
            core_axis_name=("core", "subcore"),
            dimension_semantics=(pltpu.PARALLEL,),
        )(i_hbm, o_hbm)

    return gather_kernel(tok_emb, idx2d)


def _dense_body(x_ref, pos_ref, gamma_ref, beta_ref, wt_ref, b_ref, o_ref):
    d = pos_ref.shape[2]
    x = x_ref[:, :d] + pos_ref[0]
    mean = jnp.mean(x, axis=1, keepdims=True)
    xc = x - mean
    var = jnp.mean(xc * xc, axis=1, keepdims=True)
    xn = xc * jax.lax.rsqrt(var + EPS)
    xn = xn * gamma_ref[...] + beta_ref[...]
    logits = jax.lax.dot_general(
        wt_ref[...], xn,
        (((1,), (1,)), ((), ())),
        preferred_element_type=jnp.float32,
    ) + b_ref[...]
    o_ref[...] = logits[None]


def _tc_dense(x_t, pos_emb3, gamma, beta, Wt, b_col, batch, seq, d, v,
              interpret=False):
    dx = x_t.shape[1]
    return pl.pallas_call(
        _dense_body,
        grid=(seq,),
        in_specs=[
            pl.BlockSpec((batch, dx), lambda s: (s, 0)),
            pl.BlockSpec((1, 1, d), lambda s: (s, 0, 0)),
            pl.BlockSpec((1, d), lambda s: (0, 0)),
            pl.BlockSpec((1, d), lambda s: (0, 0)),
            pl.BlockSpec((v, d), lambda s: (0, 0)),
            pl.BlockSpec((v, 1), lambda s: (0, 0)),
        ],
        out_specs=pl.BlockSpec((1, v, batch), lambda s: (s, 0, 0)),
        out_shape=jax.ShapeDtypeStruct((seq, v, batch), jnp.float32),
        compiler_params=pltpu.CompilerParams(
            dimension_semantics=("arbitrary",),
        ),
        interpret=interpret,
    )(x_t, pos_emb3, gamma, beta, Wt, b_col)


def kernel(inputs, tok_emb, pos_emb, gamma, beta, W, b):
    batch, seq = inputs.shape
    vocab, d = tok_emb.shape
    v_out = W.shape[1]
    n = batch * seq

    idx2d = inputs.T.reshape(1, n).astype(jnp.int32)
    d_pad = 128
    tok_pad = jnp.pad(tok_emb, ((0, 0), (0, d_pad - d)))
    x_t = _sc_gather(tok_pad, idx2d, n, d_pad)

    logits_t = _tc_dense(x_t, pos_emb.reshape(seq, 1, d),
                         gamma.reshape(1, d), beta.reshape(1, d), W.T,
                         b.reshape(v_out, 1), batch, seq, d, v_out)
    return jnp.transpose(logits_t, (2, 0, 1))


# speedup vs baseline: 1.1674x; 1.0212x over previous
import jax
import jax.numpy as jnp
from jax.experimental import pallas as pl
from jax.experimental.pallas import tpu as pltpu
from jax.experimental.pallas import tpu_sc as plsc

EPS = 1e-3

_GATHER_WINDOW = 128


def _sc_gather(tok_emb, idx2d, n, d):
    mesh = plsc.VectorSubcoreMesh(core_axis_name="core",
                                  subcore_axis_name="subcore")

    @pl.kernel(out_type=jax.ShapeDtypeStruct((n, d), tok_emb.dtype),
               mesh=mesh,
               scratch_types=[pltpu.VMEM_SHARED(tok_emb.shape,
                                                tok_emb.dtype),
                              pltpu.SemaphoreType.DMA])
    def gather_kernel(x_hbm, i_hbm, o_hbm, tab_ref, sem):
        @pl.when(jax.lax.axis_index("subcore") == 0)
        def _():
            pltpu.async_copy(x_hbm, tab_ref, sem).wait()

        plsc.subcore_barrier()

        def body(i_vmem, o_vmem):
            pltpu.sync_copy(tab_ref.at[i_vmem.at[0]], o_vmem)

        pltpu.emit_pipeline(
            body,
            grid=(n // _GATHER_WINDOW,),
            in_specs=[pl.BlockSpec((1, _GATHER_WINDOW),
                                   index_map=lambda i: (0, i))],
            out_specs=[pl.BlockSpec((_GATHER_WINDOW, d),
                                    index_map=lambda i: (i, 0))],
            core_axis_name=("core", "subcore"),
            dimension_semantics=(pltpu.PARALLEL,),
        )(i_hbm, o_hbm)

    return gather_kernel(tok_emb, idx2d)


def _dense_body(x_ref, pos_ref, gamma_ref, beta_ref, wt_ref, b_ref, o_ref):
    d = pos_ref.shape[2]
    x = x_ref[:, :d] + pos_ref[0]
    mean = jnp.mean(x, axis=1, keepdims=True)
    xc = x - mean
    var = jnp.mean(xc * xc, axis=1, keepdims=True)
    xn = xc * jax.lax.rsqrt(var + EPS)
    xn = xn * gamma_ref[...] + beta_ref[...]
    logits = jax.lax.dot_general(
        wt_ref[...], xn,
        (((1,), (1,)), ((), ())),
        preferred_element_type=jnp.float32,
    ) + b_ref[...]
    o_ref[...] = logits[None]


def _tc_dense(x_t, pos_emb3, gamma, beta, Wt, b_col, batch, seq, d, v,
              interpret=False):
    dx = x_t.shape[1]
    return pl.pallas_call(
        _dense_body,
        grid=(seq,),
        in_specs=[
            pl.BlockSpec((batch, dx), lambda s: (s, 0)),
            pl.BlockSpec((1, 1, d), lambda s: (s, 0, 0)),
            pl.BlockSpec((1, d), lambda s: (0, 0)),
            pl.BlockSpec((1, d), lambda s: (0, 0)),
            pl.BlockSpec((v, d), lambda s: (0, 0)),
            pl.BlockSpec((v, 1), lambda s: (0, 0)),
        ],
        out_specs=pl.BlockSpec((1, v, batch), lambda s: (s, 0, 0)),
        out_shape=jax.ShapeDtypeStruct((seq, v, batch), jnp.float32),
        compiler_params=pltpu.CompilerParams(
            dimension_semantics=("arbitrary",),
        ),
        interpret=interpret,
    )(x_t, pos_emb3, gamma, beta, Wt, b_col)


def kernel(inputs, tok_emb, pos_emb, gamma, beta, W, b):
    batch, seq = inputs.shape
    vocab, d = tok_emb.shape
    v_out = W.shape[1]
    n = batch * seq

    idx2d = inputs.T.reshape(1, n).astype(jnp.int32)
    x_t = _sc_gather(tok_emb, idx2d, n, d)

    logits_t = _tc_dense(x_t, pos_emb.reshape(seq, 1, d),
                         gamma.reshape(1, d), beta.reshape(1, d), W.T,
                         b.reshape(v_out, 1), batch, seq, d, v_out)
    return jnp.transpose(logits_t, (2, 0, 1))
